# Initial kernel scaffold; baseline (speedup 1.0000x reference)
import jax
import jax.numpy as jnp
from jax.experimental import pallas as pl


def kernel(x, edge_index, batch, W1, b1, W2, b2, W3, b3, W4, b4, g1, be1, g2, be2, g3, be3, g4, be4, Wl1, bl1, Wl2, bl2):
    raise NotImplementedError("write your pallas kernel here")


# speedup vs baseline: 6.3799x; 6.3799x over previous
import functools

import jax
import jax.numpy as jnp
from jax import lax
from jax.experimental import pallas as pl
from jax.experimental.pallas import tpu as pltpu
from jax.experimental.pallas import tpu_sc as plsc

N = 10000
E = 320000
H = 128
HH = 64
NG = 64
OUT = 64

NW = 32
CHUNK = 128
ROUNDS = 80
E_PAD = NW * ROUNDS * CHUNK
N_PAD = 10240
RPT = N_PAD // 16
DEGW = 8

_mesh = plsc.VectorSubcoreMesh(core_axis_name="c", subcore_axis_name="s",
                               num_cores=2, num_subcores=16)


@functools.partial(
    pl.kernel,
    out_type=(jax.ShapeDtypeStruct((2, N_PAD, HH), jnp.float32),
              jax.ShapeDtypeStruct((2, N_PAD, HH), jnp.float32)),
    mesh=_mesh,
    scratch_types=[
        pltpu.VMEM((ROUNDS, CHUNK), jnp.int32),
        pltpu.VMEM((ROUNDS, CHUNK), jnp.int32),
        pltpu.VMEM((CHUNK, HH), jnp.float32),
        pltpu.VMEM((CHUNK, HH), jnp.float32),
        pltpu.VMEM((CHUNK, HH), jnp.float32),
        pltpu.VMEM((CHUNK, HH), jnp.float32),
        pltpu.VMEM((RPT // 4, HH), jnp.float32),
        pltpu.VMEM_SHARED((N_PAD, HH), jnp.float32),
        pltpu.SemaphoreType.DMA,
        pltpu.SemaphoreType.DMA,
        pltpu.SemaphoreType.DMA,
        pltpu.SemaphoreType.DMA,
    ],
    compiler_params=pltpu.CompilerParams(use_tc_tiling_on_sc=False),
)
def _sc_scatter(u0_hbm, u1_hbm, src_hbm, dst_hbm, o0_hbm, o1_hbm,
                src_v, dst_v, r0, r1, r2, r3, zbuf, acc, s0, s1, s2, s3):
    cid = lax.axis_index("c")
    sid = lax.axis_index("s")
    wid = cid * 16 + sid
    pltpu.sync_copy(src_hbm.at[wid], src_v)
    pltpu.sync_copy(dst_hbm.at[wid], dst_v)

    zvec = jnp.zeros((16,), jnp.float32)

    def zbody(i, carry):
        for j in range(HH // 16):
            zbuf[i, pl.ds(j * 16, 16)] = zvec
        return carry

    lax.fori_loop(0, RPT // 4, zbody, 0)

    bufs = (r0, r1, r2, r3)
    sems = (s0, s1, s2, s3)

    for u_hbm, out_hbm in ((u0_hbm, o0_hbm), (u1_hbm, o1_hbm)):
        for q in range(4):
            pltpu.sync_copy(zbuf,
                            acc.at[pl.ds(sid * RPT + q * (RPT // 4),
                                         RPT // 4)])
        plsc.subcore_barrier()

        def gather(k, b):
            pltpu.async_copy(u_hbm.at[src_v.at[k]], bufs[b], sems[b])

        def wait_gather(k, b):
            pltpu.make_async_copy(u_hbm.at[src_v.at[k]], bufs[b],
                                  sems[b]).wait()

        def scatter(k, b):
            pltpu.async_copy(bufs[b], acc.at[dst_v.at[k]], sems[b], add=True)

        def wait_scatter(k, b):
            pltpu.make_async_copy(bufs[b], acc.at[dst_v.at[k]],
                                  sems[b]).wait()

        gather(0, 0)
        gather(1, 1)

        def body(i, carry):
            for b in range(4):
                k = 4 * i + b
                wait_gather(k, b)
                scatter(k, b)
                b2 = (b + 2) % 4

                @pl.when(k + 2 < ROUNDS)
                def _(k=k, b2=b2):
                    @pl.when(k >= 2)
                    def _():
                        wait_scatter(k - 2, b2)

                    gather(k + 2, b2)

            return carry

        lax.fori_loop(0, ROUNDS // 4, body, 0)
        for t in range(4):
            k = ROUNDS - 4 + t
            wait_scatter(k, k % 4)
        plsc.subcore_barrier()
        pltpu.sync_copy(acc.at[pl.ds(sid * RPT, RPT)],
                        out_hbm.at[cid, pl.ds(sid * RPT, RPT)])
        plsc.subcore_barrier()


@functools.partial(
    pl.kernel,
    out_type=jax.ShapeDtypeStruct((2, N_PAD, DEGW), jnp.float32),
    mesh=_mesh,
    scratch_types=[
        pltpu.VMEM((ROUNDS, CHUNK), jnp.int32),
        pltpu.VMEM((CHUNK, DEGW), jnp.float32),
        pltpu.VMEM_SHARED((N_PAD, DEGW), jnp.float32),
        pltpu.SemaphoreType.DMA,
    ],
    compiler_params=pltpu.CompilerParams(use_tc_tiling_on_sc=False),
)
def _sc_degree(dst_hbm, ones_hbm, zeros_hbm, out_hbm, dst_v, ones_v, acc, sem):
    cid = lax.axis_index("c")
    sid = lax.axis_index("s")
    wid = cid * 16 + sid
    pltpu.sync_copy(dst_hbm.at[wid], dst_v)
    pltpu.sync_copy(ones_hbm, ones_v)
    pltpu.sync_copy(zeros_hbm, acc.at[pl.ds(sid * RPT, RPT)])
    plsc.subcore_barrier()

    def body(i, carry):
        for t in range(8):
            pltpu.async_copy(ones_v, acc.at[dst_v.at[8 * i + t]], sem,
                             add=True)
        for t in range(8):
            pltpu.make_async_copy(ones_v, acc.at[dst_v.at[8 * i + t]],
                                  sem).wait()
        return carry

    lax.fori_loop(0, ROUNDS // 8, body, 0)
    plsc.subcore_barrier()
    pltpu.sync_copy(acc.at[pl.ds(sid * RPT, RPT)],
                    out_hbm.at[cid, pl.ds(sid * RPT, RPT)])


BR = 640
NBLK = N_PAD // BR


def _dinv_of(degp_ref):
    deg = degp_ref[0][:, 0:1] + degp_ref[1][:, 0:1] + 1.0
    return lax.rsqrt(deg)


def _t0_body(x_ref, w_ref, degp_ref, u0_ref, u1_ref):
    dinv = _dinv_of(degp_ref)
    r = jnp.dot(x_ref[...], w_ref[...],
                preferred_element_type=jnp.float32) * dinv
    u0_ref[...] = r[:, :HH]
    u1_ref[...] = r[:, HH:]


def _t0(x_pad, w, degp):
    return pl.pallas_call(
        _t0_body,
        grid=(NBLK,),
        in_specs=[
            pl.BlockSpec((BR, H), lambda i: (i, 0)),
            pl.BlockSpec((H, H), lambda i: (0, 0)),
            pl.BlockSpec((2, BR, DEGW), lambda i: (0, i, 0)),
        ],
        out_specs=[pl.BlockSpec((BR, HH), lambda i: (i, 0)),
                   pl.BlockSpec((BR, HH), lambda i: (i, 0))],
        out_shape=[jax.ShapeDtypeStruct((N_PAD, HH), jnp.float32),
                   jax.ShapeDtypeStruct((N_PAD, HH), jnp.float32)],
    )(x_pad, w, degp)


def _combine_c(p0_ref, p1_ref, u0_ref, u1_ref, degp_ref, b_ref, i):
    dinv = _dinv_of(degp_ref)
    agg = jnp.concatenate([p0_ref[0] + p0_ref[1] + u0_ref[...],
                           p1_ref[0] + p1_ref[1] + u1_ref[...]], axis=1)
    c = agg * dinv + b_ref[...]
    rows = lax.broadcasted_iota(jnp.int32, (BR, 1), 0) + i * BR
    return jnp.where(rows < N, c, 0.0)


def _bn_relu(c, s_sum, s_sq, g_ref, be_ref):
    mu = s_sum[...] / N
    var = s_sq[...] / N - mu * mu
    return jnp.maximum(g_ref[...] * (c - mu) * lax.rsqrt(var + 1e-5)
                       + be_ref[...], 0.0)


def _tmid_body(p0_ref, p1_ref, u0_ref, u1_ref, degp_ref, b_ref, g_ref, be_ref,
               wn_ref, o0_ref, o1_ref, c_scr, s_sum, s_sq):
    phase = pl.program_id(0)
    i = pl.program_id(1)

    @pl.when(phase == 0)
    def _():
        @pl.when(i == 0)
        def _():
            s_sum[...] = jnp.zeros_like(s_sum)
            s_sq[...] = jnp.zeros_like(s_sq)

        c = _combine_c(p0_ref, p1_ref, u0_ref, u1_ref, degp_ref, b_ref, i)
        c_scr[pl.ds(i * BR, BR), :] = c
        s_sum[...] += jnp.sum(c, axis=0, keepdims=True)
        s_sq[...] += jnp.sum(c * c, axis=0, keepdims=True)

    @pl.when(phase == 1)
    def _():
        c = c_scr[pl.ds(i * BR, BR), :]
        h = _bn_relu(c, s_sum, s_sq, g_ref, be_ref)
        rows = lax.broadcasted_iota(jnp.int32, (BR, 1), 0) + i * BR
        h = jnp.where(rows < N, h, 0.0)
        dinv = _dinv_of(degp_ref)
        r = jnp.dot(h, wn_ref[...], preferred_element_type=jnp.float32) * dinv
        o0_ref[...] = r[:, :HH]
        o1_ref[...] = r[:, HH:]


def _tmid(p0, p1, u0, u1, degp, b, g, be, wn):
    return pl.pallas_call(
        _tmid_body,
        grid=(2, NBLK),
        in_specs=[
            pl.BlockSpec((2, BR, HH), lambda p_, i: (0, i, 0)),
            pl.BlockSpec((2, BR, HH), lambda p_, i: (0, i, 0)),
            pl.BlockSpec((BR, HH), lambda p_, i: (i, 0)),
            pl.BlockSpec((BR, HH), lambda p_, i: (i, 0)),
            pl.BlockSpec((2, BR, DEGW), lambda p_, i: (0, i, 0)),
            pl.BlockSpec((H,), lambda p_, i: (0,)),
            pl.BlockSpec((H,), lambda p_, i: (0,)),
            pl.BlockSpec((H,), lambda p_, i: (0,)),
            pl.BlockSpec((H, H), lambda p_, i: (0, 0)),
        ],
        out_specs=[pl.BlockSpec((BR, HH), lambda p_, i: (i, 0)),
                   pl.BlockSpec((BR, HH), lambda p_, i: (i, 0))],
        out_shape=[jax.ShapeDtypeStruct((N_PAD, HH), jnp.float32),
                   jax.ShapeDtypeStruct((N_PAD, HH), jnp.float32)],
        scratch_shapes=[
            pltpu.VMEM((N_PAD, H), jnp.float32),
            pltpu.VMEM((1, H), jnp.float32),
            pltpu.VMEM((1, H), jnp.float32),
        ],
    )(p0, p1, u0, u1, degp, b, g, be, wn)


def _tfinal_body(p0_ref, p1_ref, u0_ref, u1_ref, degp_ref, b_ref, g_ref,
                 be_ref, bat_ref, wl1_ref, bl1_ref, wl2_ref, bl2_ref, out_ref,
                 c_scr, s_sum, s_sq, psum, pcnt):
    phase = pl.program_id(0)
    i = pl.program_id(1)

    @pl.when(phase == 0)
    def _():
        @pl.when(i == 0)
        def _():
            s_sum[...] = jnp.zeros_like(s_sum)
            s_sq[...] = jnp.zeros_like(s_sq)
            psum[...] = jnp.zeros_like(psum)
            pcnt[...] = jnp.zeros_like(pcnt)

        c = _combine_c(p0_ref, p1_ref, u0_ref, u1_ref, degp_ref, b_ref, i)
        c_scr[pl.ds(i * BR, BR), :] = c
        s_sum[...] += jnp.sum(c, axis=0, keepdims=True)
        s_sq[...] += jnp.sum(c * c, axis=0, keepdims=True)

    @pl.when(phase == 1)
    def _():
        c = c_scr[pl.ds(i * BR, BR), :]
        h = _bn_relu(c, s_sum, s_sq, g_ref, be_ref)
        gids = lax.broadcasted_iota(jnp.int32, (NG, BR), 0)
        pmat = (bat_ref[...] == gids).astype(jnp.float32)
        psum[...] += jnp.dot(pmat, h, preferred_element_type=jnp.float32)
        pcnt[...] += jnp.sum(pmat, axis=1, keepdims=True)

        @pl.when(i == NBLK - 1)
        def _():
            pooled = psum[...] / jnp.maximum(pcnt[...], 1.0)
            z = jnp.maximum(jnp.dot(pooled, wl1_ref[...],
                                    preferred_element_type=jnp.float32)
                            + bl1_ref[...], 0.0)
            out_ref[...] = jnp.dot(z, wl2_ref[...],
                                   preferred_element_type=jnp.float32) \
                + bl2_ref[...]


def _tfinal(p0, p1, u0, u1, degp, b, g, be, bat, wl1, bl1, wl2, bl2):
    return pl.pallas_call(
        _tfinal_body,
        grid=(2, NBLK),
        in_specs=[
            pl.BlockSpec((2, BR, HH), lambda p_, i: (0, i, 0)),
            pl.BlockSpec((2, BR, HH), lambda p_, i: (0, i, 0)),
            pl.BlockSpec((BR, HH), lambda p_, i: (i, 0)),
            pl.BlockSpec((BR, HH), lambda p_, i: (i, 0)),
            pl.BlockSpec((2, BR, DEGW), lambda p_, i: (0, i, 0)),
            pl.BlockSpec((H,), lambda p_, i: (0,)),
            pl.BlockSpec((H,), lambda p_, i: (0,)),
            pl.BlockSpec((H,), lambda p_, i: (0,)),
            pl.BlockSpec((1, BR), lambda p_, i: (0, i)),
            pl.BlockSpec((H, H // 2), lambda p_, i: (0, 0)),
            pl.BlockSpec((H // 2,), lambda p_, i: (0,)),
            pl.BlockSpec((H // 2, OUT), lambda p_, i: (0, 0)),
            pl.BlockSpec((OUT,), lambda p_, i: (0,)),
        ],
        out_specs=pl.BlockSpec((NG, OUT), lambda p_, i: (0, 0)),
        out_shape=jax.ShapeDtypeStruct((NG, OUT), jnp.float32),
        scratch_shapes=[
            pltpu.VMEM((N_PAD, H), jnp.float32),
            pltpu.VMEM((1, H), jnp.float32),
            pltpu.VMEM((1, H), jnp.float32),
            pltpu.VMEM((NG, H), jnp.float32),
            pltpu.VMEM((NG, 1), jnp.float32),
        ],
    )(p0, p1, u0, u1, degp, b, g, be, bat, wl1, bl1, wl2, bl2)


def kernel(x, edge_index, batch, W1, b1, W2, b2, W3, b3, W4, b4,
           g1, be1, g2, be2, g3, be3, g4, be4, Wl1, bl1, Wl2, bl2):
    src = edge_index[0]
    dst = edge_index[1]
    pad = jnp.full((E_PAD - E,), N, dtype=jnp.int32)
    src3 = jnp.concatenate([src, pad]).reshape(NW, ROUNDS, CHUNK)
    dst3 = jnp.concatenate([dst, pad]).reshape(NW, ROUNDS, CHUNK)

    x_pad = jnp.pad(x, ((0, N_PAD - N), (0, 0)))
    bat = jnp.pad(batch, (0, N_PAD - N), constant_values=NG + 1).reshape(1, N_PAD)

    zeros_d = jnp.zeros((RPT, DEGW), jnp.float32)
    ones_d = jnp.zeros((CHUNK, DEGW), jnp.float32).at[:, 0].set(1.0)

    degp = _sc_degree(dst3, ones_d, zeros_d)

    u0, u1 = _t0(x_pad, W1, degp)
    p0, p1 = _sc_scatter(u0, u1, src3, dst3)
    u0, u1 = _tmid(p0, p1, u0, u1, degp, b1, g1, be1, W2)
    p0, p1 = _sc_scatter(u0, u1, src3, dst3)
    u0, u1 = _tmid(p0, p1, u0, u1, degp, b2, g2, be2, W3)
    p0, p1 = _sc_scatter(u0, u1, src3, dst3)
    u0, u1 = _tmid(p0, p1, u0, u1, degp, b3, g3, be3, W4)
    p0, p1 = _sc_scatter(u0, u1, src3, dst3)
    return _tfinal(p0, p1, u0, u1, degp, b4, g4, be4, bat, Wl1, bl1, Wl2, bl2)


# speedup vs baseline: 18.1271x; 2.8413x over previous
import functools

import jax
import jax.numpy as jnp
from jax import lax
from jax.experimental import pallas as pl
from jax.experimental.pallas import tpu as pltpu
from jax.experimental.pallas import tpu_sc as plsc

N = 10000
E = 320000
H = 128
HH = 64
NG = 64
OUT = 64

NW = 32
CHUNK = 128
ROUNDS = 80
E_PAD = NW * ROUNDS * CHUNK
N_PAD = 10240
RPT = N_PAD // 16
DEGW = 8

_mesh = plsc.VectorSubcoreMesh(core_axis_name="c", subcore_axis_name="s",
                               num_cores=2, num_subcores=16)


RNDS = 2 * ROUNDS
GRPS = RNDS // 8


@functools.partial(
    pl.kernel,
    out_type=(jax.ShapeDtypeStruct((N_PAD, HH), jnp.float32),
              jax.ShapeDtypeStruct((N_PAD, HH), jnp.float32)),
    mesh=_mesh,
    scratch_types=[
        pltpu.VMEM((2, 8, CHUNK), jnp.int32),
        pltpu.VMEM((2, 8, CHUNK), jnp.int32),
        pltpu.VMEM((CHUNK, HH), jnp.float32),
        pltpu.VMEM((CHUNK, HH), jnp.float32),
        pltpu.VMEM_SHARED((N_PAD, HH), jnp.float32),
        pltpu.VMEM_SHARED((N_PAD, HH), jnp.float32),
        pltpu.SemaphoreType.DMA,
        pltpu.SemaphoreType.DMA,
        pltpu.SemaphoreType.DMA,
        pltpu.SemaphoreType.DMA,
    ],
    compiler_params=pltpu.CompilerParams(use_tc_tiling_on_sc=False),
)
def _sc_scatter(u0_hbm, u1_hbm, src_hbm, dst_hbm, o0_hbm, o1_hbm,
                src_blk, dst_blk, r0, r1, ustage, acc, s0, s1, si, di):
    cid = lax.axis_index("c")
    sid = lax.axis_index("s")

    bufs = (r0, r1)
    sems = (s0, s1)
    zvec = jnp.zeros((16,), jnp.float32)

    def full_pass(u_hbm, out_hbm):
        def zbody(i, carry):
            for j in range(HH // 16):
                r0[i, pl.ds(j * 16, 16)] = zvec
            return carry

        lax.fori_loop(0, CHUNK, zbody, 0)
        for q in range(RPT // CHUNK):
            pltpu.sync_copy(r0, acc.at[pl.ds(sid * RPT + q * CHUNK, CHUNK)])
        pltpu.sync_copy(u_hbm.at[pl.ds(sid * RPT, RPT)],
                        ustage.at[pl.ds(sid * RPT, RPT)])
        plsc.subcore_barrier()

        def idx_fire(g, q):
            pltpu.async_copy(src_hbm.at[sid, pl.ds(g * 8, 8)],
                             src_blk.at[q], si)
            pltpu.async_copy(dst_hbm.at[sid, pl.ds(g * 8, 8)],
                             dst_blk.at[q], di)

        def idx_wait(g, q):
            pltpu.make_async_copy(src_hbm.at[sid, pl.ds(g * 8, 8)],
                                  src_blk.at[q], si).wait()
            pltpu.make_async_copy(dst_hbm.at[sid, pl.ds(g * 8, 8)],
                                  dst_blk.at[q], di).wait()

        def gather(k, q, r, b):
            pltpu.async_copy(ustage.at[src_blk.at[q, r]], bufs[b], sems[b])

        def wait_gather(k, q, r, b):
            pltpu.make_async_copy(ustage.at[src_blk.at[q, r]], bufs[b],
                                  sems[b]).wait()

        def scatter(k, q, r, b):
            pltpu.async_copy(bufs[b], acc.at[dst_blk.at[q, r]], sems[b],
                             add=True)

        def wait_scatter(k, q, r, b):
            pltpu.make_async_copy(bufs[b], acc.at[dst_blk.at[q, r]],
                                  sems[b]).wait()

        pltpu.sync_copy(src_hbm.at[sid, pl.ds(0, 8)], src_blk.at[0])
        pltpu.sync_copy(dst_hbm.at[sid, pl.ds(0, 8)], dst_blk.at[0])
        idx_fire(1, 1)
        gather(0, 0, 0, 0)

        def body(i, carry):
            for kk in range(16):
                k = 16 * i + kk
                q = kk // 8
                r = kk % 8
                b = kk % 2
                wait_gather(k, q, r, b)
                scatter(k, q, r, b)

                if kk == 1:
                    @pl.when(i >= 1)
                    def _(i=i):
                        idx_fire(2 * i + 1, 1)
                if kk == 9:
                    @pl.when(i < (GRPS - 2) // 2)
                    def _(i=i):
                        idx_fire(2 * i + 2, 0)
                if kk == 7:
                    idx_wait(2 * i + 1, 1)

                nq = (kk + 1) // 8 % 2
                nr = (kk + 1) % 8
                nb = (kk + 1) % 2
                if kk == 15:
                    @pl.when(i < (RNDS // 16) - 1)
                    def _(i=i, k=k):
                        idx_wait(2 * i + 2, 0)
                        wait_scatter(k - 1, 1, 7, nb)
                        gather(k + 1, 0, 0, nb)
                else:
                    @pl.when(k >= 1)
                    def _(k=k, q=q, r=r, nb=nb):
                        wait_scatter(k - 1, q, r, nb)

                    gather(k + 1, nq, nr, nb)

            return carry

        lax.fori_loop(0, RNDS // 16, body, 0)
        wait_scatter(RNDS - 2, 1, 6, 0)
        wait_scatter(RNDS - 1, 1, 7, 1)
        plsc.subcore_barrier()
        pltpu.sync_copy(acc.at[pl.ds(sid * RPT, RPT)],
                        out_hbm.at[pl.ds(sid * RPT, RPT)])

    @pl.when(cid == 0)
    def _():
        full_pass(u0_hbm, o0_hbm)

    @pl.when(cid == 1)
    def _():
        full_pass(u1_hbm, o1_hbm)


@functools.partial(
    pl.kernel,
    out_type=jax.ShapeDtypeStruct((2, N_PAD, DEGW), jnp.float32),
    mesh=_mesh,
    scratch_types=[
        pltpu.VMEM((ROUNDS, CHUNK), jnp.int32),
        pltpu.VMEM((CHUNK, DEGW), jnp.float32),
        pltpu.VMEM_SHARED((N_PAD, DEGW), jnp.float32),
        pltpu.SemaphoreType.DMA,
    ],
    compiler_params=pltpu.CompilerParams(use_tc_tiling_on_sc=False),
)
def _sc_degree(dst_hbm, ones_hbm, zeros_hbm, out_hbm, dst_v, ones_v, acc, sem):
    cid = lax.axis_index("c")
    sid = lax.axis_index("s")
    pltpu.sync_copy(dst_hbm.at[sid, pl.ds(cid * ROUNDS, ROUNDS)], dst_v)
    pltpu.sync_copy(ones_hbm, ones_v)
    pltpu.sync_copy(zeros_hbm, acc.at[pl.ds(sid * RPT, RPT)])
    plsc.subcore_barrier()

    def body(i, carry):
        for t in range(8):
            pltpu.async_copy(ones_v, acc.at[dst_v.at[8 * i + t]], sem,
                             add=True)
        for t in range(8):
            pltpu.make_async_copy(ones_v, acc.at[dst_v.at[8 * i + t]],
                                  sem).wait()
        return carry

    lax.fori_loop(0, ROUNDS // 8, body, 0)
    plsc.subcore_barrier()
    pltpu.sync_copy(acc.at[pl.ds(sid * RPT, RPT)],
                    out_hbm.at[cid, pl.ds(sid * RPT, RPT)])


BR = 2560
NBLK = N_PAD // BR


def _dinv_of(degp_ref):
    deg = degp_ref[0][:, 0:1] + degp_ref[1][:, 0:1] + 1.0
    return lax.rsqrt(deg)


def _t0_body(x_ref, w_ref, degp_ref, u0_ref, u1_ref):
    dinv = _dinv_of(degp_ref)
    r = jnp.dot(x_ref[...], w_ref[...],
                preferred_element_type=jnp.float32) * dinv
    u0_ref[...] = r[:, :HH]
    u1_ref[...] = r[:, HH:]


def _t0(x_pad, w, degp):
    return pl.pallas_call(
        _t0_body,
        grid=(NBLK,),
        in_specs=[
            pl.BlockSpec((BR, H), lambda i: (i, 0)),
            pl.BlockSpec((H, H), lambda i: (0, 0)),
            pl.BlockSpec((2, BR, DEGW), lambda i: (0, i, 0)),
        ],
        out_specs=[pl.BlockSpec((BR, HH), lambda i: (i, 0)),
                   pl.BlockSpec((BR, HH), lambda i: (i, 0))],
        out_shape=[jax.ShapeDtypeStruct((N_PAD, HH), jnp.float32),
                   jax.ShapeDtypeStruct((N_PAD, HH), jnp.float32)],
    )(x_pad, w, degp)


def _combine_c(p0_ref, p1_ref, u0_ref, u1_ref, degp_ref, b_ref, i):
    dinv = _dinv_of(degp_ref)
    agg = jnp.concatenate([p0_ref[...] + u0_ref[...],
                           p1_ref[...] + u1_ref[...]], axis=1)
    c = agg * dinv + b_ref[...]
    rows = lax.broadcasted_iota(jnp.int32, (BR, 1), 0) + i * BR
    return jnp.where(rows < N, c, 0.0)


def _bn_relu(c, s_sum, s_sq, g_ref, be_ref):
    mu = s_sum[...] / N
    var = s_sq[...] / N - mu * mu
    return jnp.maximum(g_ref[...] * (c - mu) * lax.rsqrt(var + 1e-5)
                       + be_ref[...], 0.0)


def _tmid_body(p0_ref, p1_ref, u0_ref, u1_ref, degp_ref, b_ref, g_ref, be_ref,
               wn_ref, o0_ref, o1_ref, c_scr, s_sum, s_sq):
    phase = pl.program_id(0)
    i = pl.program_id(1)

    @pl.when(phase == 0)
    def _():
        @pl.when(i == 0)
        def _():
            s_sum[...] = jnp.zeros_like(s_sum)
            s_sq[...] = jnp.zeros_like(s_sq)

        c = _combine_c(p0_ref, p1_ref, u0_ref, u1_ref, degp_ref, b_ref, i)
        c_scr[pl.ds(i * BR, BR), :] = c
        s_sum[...] += jnp.sum(c, axis=0, keepdims=True)
        s_sq[...] += jnp.sum(c * c, axis=0, keepdims=True)

    @pl.when(phase == 1)
    def _():
        c = c_scr[pl.ds(i * BR, BR), :]
        h = _bn_relu(c, s_sum, s_sq, g_ref, be_ref)
        rows = lax.broadcasted_iota(jnp.int32, (BR, 1), 0) + i * BR
        h = jnp.where(rows < N, h, 0.0)
        dinv = _dinv_of(degp_ref)
        r = jnp.dot(h, wn_ref[...], preferred_element_type=jnp.float32) * dinv
        o0_ref[...] = r[:, :HH]
        o1_ref[...] = r[:, HH:]


def _tmid(p0, p1, u0, u1, degp, b, g, be, wn):
    return pl.pallas_call(
        _tmid_body,
        grid=(2, NBLK),
        in_specs=[
            pl.BlockSpec((BR, HH), lambda p_, i: (i, 0)),
            pl.BlockSpec((BR, HH), lambda p_, i: (i, 0)),
            pl.BlockSpec((BR, HH), lambda p_, i: (i, 0)),
            pl.BlockSpec((BR, HH), lambda p_, i: (i, 0)),
            pl.BlockSpec((2, BR, DEGW), lambda p_, i: (0, i, 0)),
            pl.BlockSpec((H,), lambda p_, i: (0,)),
            pl.BlockSpec((H,), lambda p_, i: (0,)),
            pl.BlockSpec((H,), lambda p_, i: (0,)),
            pl.BlockSpec((H, H), lambda p_, i: (0, 0)),
        ],
        out_specs=[pl.BlockSpec((BR, HH), lambda p_, i: (i, 0)),
                   pl.BlockSpec((BR, HH), lambda p_, i: (i, 0))],
        out_shape=[jax.ShapeDtypeStruct((N_PAD, HH), jnp.float32),
                   jax.ShapeDtypeStruct((N_PAD, HH), jnp.float32)],
        scratch_shapes=[
            pltpu.VMEM((N_PAD, H), jnp.float32),
            pltpu.VMEM((1, H), jnp.float32),
            pltpu.VMEM((1, H), jnp.float32),
        ],
    )(p0, p1, u0, u1, degp, b, g, be, wn)


def _tfinal_body(p0_ref, p1_ref, u0_ref, u1_ref, degp_ref, b_ref, g_ref,
                 be_ref, bat_ref, wl1_ref, bl1_ref, wl2_ref, bl2_ref, out_ref,
                 c_scr, s_sum, s_sq, psum, pcnt):
    phase = pl.program_id(0)
    i = pl.program_id(1)

    @pl.when(phase == 0)
    def _():
        @pl.when(i == 0)
        def _():
            s_sum[...] = jnp.zeros_like(s_sum)
            s_sq[...] = jnp.zeros_like(s_sq)
            psum[...] = jnp.zeros_like(psum)
            pcnt[...] = jnp.zeros_like(pcnt)

        c = _combine_c(p0_ref, p1_ref, u0_ref, u1_ref, degp_ref, b_ref, i)
        c_scr[pl.ds(i * BR, BR), :] = c
        s_sum[...] += jnp.sum(c, axis=0, keepdims=True)
        s_sq[...] += jnp.sum(c * c, axis=0, keepdims=True)

    @pl.when(phase == 1)
    def _():
        c = c_scr[pl.ds(i * BR, BR), :]
        h = _bn_relu(c, s_sum, s_sq, g_ref, be_ref)
        gids = lax.broadcasted_iota(jnp.int32, (NG, BR), 0)
        pmat = (bat_ref[...] == gids).astype(jnp.float32)
        psum[...] += jnp.dot(pmat, h, preferred_element_type=jnp.float32)
        pcnt[...] += jnp.sum(pmat, axis=1, keepdims=True)

        @pl.when(i == NBLK - 1)
        def _():
            pooled = psum[...] / jnp.maximum(pcnt[...], 1.0)
            z = jnp.maximum(jnp.dot(pooled, wl1_ref[...],
                                    preferred_element_type=jnp.float32)
                            + bl1_ref[...], 0.0)
            out_ref[...] = jnp.dot(z, wl2_ref[...],
                                   preferred_element_type=jnp.float32) \
                + bl2_ref[...]


def _tfinal(p0, p1, u0, u1, degp, b, g, be, bat, wl1, bl1, wl2, bl2):
    return pl.pallas_call(
        _tfinal_body,
        grid=(2, NBLK),
        in_specs=[
            pl.BlockSpec((BR, HH), lambda p_, i: (i, 0)),
            pl.BlockSpec((BR, HH), lambda p_, i: (i, 0)),
            pl.BlockSpec((BR, HH), lambda p_, i: (i, 0)),
            pl.BlockSpec((BR, HH), lambda p_, i: (i, 0)),
            pl.BlockSpec((2, BR, DEGW), lambda p_, i: (0, i, 0)),
            pl.BlockSpec((H,), lambda p_, i: (0,)),
            pl.BlockSpec((H,), lambda p_, i: (0,)),
            pl.BlockSpec((H,), lambda p_, i: (0,)),
            pl.BlockSpec((1, BR), lambda p_, i: (0, i)),
            pl.BlockSpec((H, H // 2), lambda p_, i: (0, 0)),
            pl.BlockSpec((H // 2,), lambda p_, i: (0,)),
            pl.BlockSpec((H // 2, OUT), lambda p_, i: (0, 0)),
            pl.BlockSpec((OUT,), lambda p_, i: (0,)),
        ],
        out_specs=pl.BlockSpec((NG, OUT), lambda p_, i: (0, 0)),
        out_shape=jax.ShapeDtypeStruct((NG, OUT), jnp.float32),
        scratch_shapes=[
            pltpu.VMEM((N_PAD, H), jnp.float32),
            pltpu.VMEM((1, H), jnp.float32),
            pltpu.VMEM((1, H), jnp.float32),
            pltpu.VMEM((NG, H), jnp.float32),
            pltpu.VMEM((NG, 1), jnp.float32),
        ],
    )(p0, p1, u0, u1, degp, b, g, be, bat, wl1, bl1, wl2, bl2)


def kernel(x, edge_index, batch, W1, b1, W2, b2, W3, b3, W4, b4,
           g1, be1, g2, be2, g3, be3, g4, be4, Wl1, bl1, Wl2, bl2):
    src = edge_index[0]
    dst = edge_index[1]
    pad = jnp.full((E_PAD - E,), N, dtype=jnp.int32)
    src_flat = jnp.concatenate([src, pad])
    dst_flat = jnp.concatenate([dst, pad])
    src3 = src_flat.reshape(16, RNDS, CHUNK)
    dst3 = dst_flat.reshape(16, RNDS, CHUNK)

    x_pad = jnp.pad(x, ((0, N_PAD - N), (0, 0)))
    bat = jnp.pad(batch, (0, N_PAD - N), constant_values=NG + 1).reshape(1, N_PAD)

    zeros_d = jnp.zeros((RPT, DEGW), jnp.float32)
    ones_d = jnp.zeros((CHUNK, DEGW), jnp.float32).at[:, 0].set(1.0)

    degp = _sc_degree(dst3, ones_d, zeros_d)

    u0, u1 = _t0(x_pad, W1, degp)
    p0, p1 = _sc_scatter(u0, u1, src3, dst3)
    u0, u1 = _tmid(p0, p1, u0, u1, degp, b1, g1, be1, W2)
    p0, p1 = _sc_scatter(u0, u1, src3, dst3)
    u0, u1 = _tmid(p0, p1, u0, u1, degp, b2, g2, be2, W3)
    p0, p1 = _sc_scatter(u0, u1, src3, dst3)
    u0, u1 = _tmid(p0, p1, u0, u1, degp, b3, g3, be3, W4)
    p0, p1 = _sc_scatter(u0, u1, src3, dst3)
    return _tfinal(p0, p1, u0, u1, degp, b4, g4, be4, bat, Wl1, bl1, Wl2, bl2)


# speedup vs baseline: 20.5256x; 1.1323x over previous
import functools

import jax
import jax.numpy as jnp
from jax import lax
from jax.experimental import pallas as pl
from jax.experimental.pallas import tpu as pltpu
from jax.experimental.pallas import tpu_sc as plsc

N = 10000
E = 320000
H = 128
HH = 64
NG = 64
OUT = 64

NW = 32
CHUNK = 128
ROUNDS = 80
E_PAD = NW * ROUNDS * CHUNK
N_PAD = 10240
RPT = N_PAD // 16
DEGW = 8

_mesh = plsc.VectorSubcoreMesh(core_axis_name="c", subcore_axis_name="s",
                               num_cores=2, num_subcores=16)


RNDS = 2 * ROUNDS
GRPS = RNDS // 8


@functools.partial(
    pl.kernel,
    out_type=jax.ShapeDtypeStruct((N_PAD, H), jnp.float32),
    mesh=_mesh,
    scratch_types=[
        pltpu.VMEM((2, 8, CHUNK), jnp.int32),
        pltpu.VMEM((2, 8, CHUNK), jnp.int32),
        pltpu.VMEM((CHUNK, HH), jnp.float32),
        pltpu.VMEM((CHUNK, HH), jnp.float32),
        pltpu.VMEM_SHARED((N_PAD, HH), jnp.float32),
        pltpu.VMEM_SHARED((N_PAD, HH), jnp.float32),
        pltpu.SemaphoreType.DMA,
        pltpu.SemaphoreType.DMA,
        pltpu.SemaphoreType.DMA,
        pltpu.SemaphoreType.DMA,
    ],
    compiler_params=pltpu.CompilerParams(use_tc_tiling_on_sc=False),
)
def _sc_scatter(u_hbm, src_hbm, dst_hbm, out_hbm,
                src_blk, dst_blk, r0, r1, ustage, acc, s0, s1, si, di):
    cid = lax.axis_index("c")
    sid = lax.axis_index("s")

    bufs = (r0, r1)
    sems = (s0, s1)
    zvec = jnp.zeros((16,), jnp.float32)

    def full_pass():
        def zbody(i, carry):
            for j in range(HH // 16):
                r0[i, pl.ds(j * 16, 16)] = zvec
            return carry

        lax.fori_loop(0, CHUNK, zbody, 0)
        for q in range(RPT // CHUNK):
            pltpu.sync_copy(r0, acc.at[pl.ds(sid * RPT + q * CHUNK, CHUNK)])
        pltpu.sync_copy(u_hbm.at[pl.ds(sid * RPT, RPT), pl.ds(cid * HH, HH)],
                        ustage.at[pl.ds(sid * RPT, RPT)])
        plsc.subcore_barrier()

        def idx_fire(g, q):
            pltpu.async_copy(src_hbm.at[sid, pl.ds(g * 8, 8)],
                             src_blk.at[q], si)
            pltpu.async_copy(dst_hbm.at[sid, pl.ds(g * 8, 8)],
                             dst_blk.at[q], di)

        def idx_wait(g, q):
            pltpu.make_async_copy(src_hbm.at[sid, pl.ds(g * 8, 8)],
                                  src_blk.at[q], si).wait()
            pltpu.make_async_copy(dst_hbm.at[sid, pl.ds(g * 8, 8)],
                                  dst_blk.at[q], di).wait()

        def gather(k, q, r, b):
            pltpu.async_copy(ustage.at[src_blk.at[q, r]], bufs[b], sems[b])

        def wait_gather(k, q, r, b):
            pltpu.make_async_copy(ustage.at[src_blk.at[q, r]], bufs[b],
                                  sems[b]).wait()

        def scatter(k, q, r, b):
            pltpu.async_copy(bufs[b], acc.at[dst_blk.at[q, r]], sems[b],
                             add=True)

        def wait_scatter(k, q, r, b):
            pltpu.make_async_copy(bufs[b], acc.at[dst_blk.at[q, r]],
                                  sems[b]).wait()

        pltpu.sync_copy(src_hbm.at[sid, pl.ds(0, 8)], src_blk.at[0])
        pltpu.sync_copy(dst_hbm.at[sid, pl.ds(0, 8)], dst_blk.at[0])
        idx_fire(1, 1)
        gather(0, 0, 0, 0)

        def body(i, carry):
            for kk in range(16):
                k = 16 * i + kk
                q = kk // 8
                r = kk % 8
                b = kk % 2
                wait_gather(k, q, r, b)
                scatter(k, q, r, b)

                if kk == 1:
                    @pl.when(i >= 1)
                    def _(i=i):
                        idx_fire(2 * i + 1, 1)
                if kk == 9:
                    @pl.when(i < (GRPS - 2) // 2)
                    def _(i=i):
                        idx_fire(2 * i + 2, 0)
                if kk == 7:
                    idx_wait(2 * i + 1, 1)

                nq = (kk + 1) // 8 % 2
                nr = (kk + 1) % 8
                nb = (kk + 1) % 2
                if kk == 15:
                    @pl.when(i < (RNDS // 16) - 1)
                    def _(i=i, k=k):
                        idx_wait(2 * i + 2, 0)
                        wait_scatter(k - 1, 1, 7, nb)
                        gather(k + 1, 0, 0, nb)
                else:
                    @pl.when(k >= 1)
                    def _(k=k, q=q, r=r, nb=nb):
                        wait_scatter(k - 1, q, r, nb)

                    gather(k + 1, nq, nr, nb)

            return carry

        lax.fori_loop(0, RNDS // 16, body, 0)
        wait_scatter(RNDS - 2, 1, 6, 0)
        wait_scatter(RNDS - 1, 1, 7, 1)
        plsc.subcore_barrier()
        pltpu.sync_copy(acc.at[pl.ds(sid * RPT, RPT)],
                        out_hbm.at[pl.ds(sid * RPT, RPT), pl.ds(cid * HH, HH)])

    full_pass()


@functools.partial(
    pl.kernel,
    out_type=jax.ShapeDtypeStruct((2, N_PAD, DEGW), jnp.float32),
    mesh=_mesh,
    scratch_types=[
        pltpu.VMEM((ROUNDS, CHUNK), jnp.int32),
        pltpu.VMEM((CHUNK, DEGW), jnp.float32),
        pltpu.VMEM_SHARED((N_PAD, DEGW), jnp.float32),
        pltpu.SemaphoreType.DMA,
    ],
    compiler_params=pltpu.CompilerParams(use_tc_tiling_on_sc=False),
)
def _sc_degree(dst_hbm, ones_hbm, zeros_hbm, out_hbm, dst_v, ones_v, acc, sem):
    cid = lax.axis_index("c")
    sid = lax.axis_index("s")
    pltpu.sync_copy(dst_hbm.at[sid, pl.ds(cid * ROUNDS, ROUNDS)], dst_v)
    pltpu.sync_copy(ones_hbm, ones_v)
    pltpu.sync_copy(zeros_hbm, acc.at[pl.ds(sid * RPT, RPT)])
    plsc.subcore_barrier()

    def body(i, carry):
        for t in range(8):
            pltpu.async_copy(ones_v, acc.at[dst_v.at[8 * i + t]], sem,
                             add=True)
        for t in range(8):
            pltpu.make_async_copy(ones_v, acc.at[dst_v.at[8 * i + t]],
                                  sem).wait()
        return carry

    lax.fori_loop(0, ROUNDS // 8, body, 0)
    plsc.subcore_barrier()
    pltpu.sync_copy(acc.at[pl.ds(sid * RPT, RPT)],
                    out_hbm.at[cid, pl.ds(sid * RPT, RPT)])


BR = 2560
NBLK = N_PAD // BR


def _dinv_of(degp_ref):
    deg = degp_ref[0][:, 0:1] + degp_ref[1][:, 0:1] + 1.0
    return lax.rsqrt(deg)


def _t0_body(x_ref, w_ref, degp_ref, u_ref):
    dinv = _dinv_of(degp_ref)
    u_ref[...] = jnp.dot(x_ref[...], w_ref[...],
                         preferred_element_type=jnp.float32) * dinv


def _t0(x_pad, w, degp):
    return pl.pallas_call(
        _t0_body,
        grid=(NBLK,),
        in_specs=[
            pl.BlockSpec((BR, H), lambda i: (i, 0)),
            pl.BlockSpec((H, H), lambda i: (0, 0)),
            pl.BlockSpec((2, BR, DEGW), lambda i: (0, i, 0)),
        ],
        out_specs=pl.BlockSpec((BR, H), lambda i: (i, 0)),
        out_shape=jax.ShapeDtypeStruct((N_PAD, H), jnp.float32),
    )(x_pad, w, degp)


def _combine_c(p_ref, u_ref, degp_ref, b_ref, i):
    dinv = _dinv_of(degp_ref)
    c = (p_ref[...] + u_ref[...]) * dinv + b_ref[...]
    rows = lax.broadcasted_iota(jnp.int32, (BR, 1), 0) + i * BR
    return jnp.where(rows < N, c, 0.0)


def _bn_relu(c, s_sum, s_sq, g_ref, be_ref):
    mu = s_sum[...] / N
    var = s_sq[...] / N - mu * mu
    return jnp.maximum(g_ref[...] * (c - mu) * lax.rsqrt(var + 1e-5)
                       + be_ref[...], 0.0)


def _tmid_body(p_ref, u_ref, degp_ref, b_ref, g_ref, be_ref,
               wn_ref, o_ref, c_scr, s_sum, s_sq):
    phase = pl.program_id(0)
    i = pl.program_id(1)

    @pl.when(phase == 0)
    def _():
        @pl.when(i == 0)
        def _():
            s_sum[...] = jnp.zeros_like(s_sum)
            s_sq[...] = jnp.zeros_like(s_sq)

        c = _combine_c(p_ref, u_ref, degp_ref, b_ref, i)
        c_scr[pl.ds(i * BR, BR), :] = c
        s_sum[...] += jnp.sum(c, axis=0, keepdims=True)
        s_sq[...] += jnp.sum(c * c, axis=0, keepdims=True)

    @pl.when(phase == 1)
    def _():
        c = c_scr[pl.ds(i * BR, BR), :]
        h = _bn_relu(c, s_sum, s_sq, g_ref, be_ref)
        rows = lax.broadcasted_iota(jnp.int32, (BR, 1), 0) + i * BR
        h = jnp.where(rows < N, h, 0.0)
        dinv = _dinv_of(degp_ref)
        o_ref[...] = jnp.dot(h, wn_ref[...],
                             preferred_element_type=jnp.float32) * dinv


def _tmid(p, u, degp, b, g, be, wn):
    return pl.pallas_call(
        _tmid_body,
        grid=(2, NBLK),
        in_specs=[
            pl.BlockSpec((BR, H), lambda p_, i: (i, 0)),
            pl.BlockSpec((BR, H), lambda p_, i: (i, 0)),
            pl.BlockSpec((2, BR, DEGW), lambda p_, i: (0, i, 0)),
            pl.BlockSpec((H,), lambda p_, i: (0,)),
            pl.BlockSpec((H,), lambda p_, i: (0,)),
            pl.BlockSpec((H,), lambda p_, i: (0,)),
            pl.BlockSpec((H, H), lambda p_, i: (0, 0)),
        ],
        out_specs=pl.BlockSpec((BR, H), lambda p_, i: (i, 0)),
        out_shape=jax.ShapeDtypeStruct((N_PAD, H), jnp.float32),
        scratch_shapes=[
            pltpu.VMEM((N_PAD, H), jnp.float32),
            pltpu.VMEM((1, H), jnp.float32),
            pltpu.VMEM((1, H), jnp.float32),
        ],
    )(p, u, degp, b, g, be, wn)


def _tfinal_body(p_ref, u_ref, degp_ref, b_ref, g_ref,
                 be_ref, bat_ref, wl1_ref, bl1_ref, wl2_ref, bl2_ref, out_ref,
                 c_scr, s_sum, s_sq, psum, pcnt):
    phase = pl.program_id(0)
    i = pl.program_id(1)

    @pl.when(phase == 0)
    def _():
        @pl.when(i == 0)
        def _():
            s_sum[...] = jnp.zeros_like(s_sum)
            s_sq[...] = jnp.zeros_like(s_sq)
            psum[...] = jnp.zeros_like(psum)
            pcnt[...] = jnp.zeros_like(pcnt)

        c = _combine_c(p_ref, u_ref, degp_ref, b_ref, i)
        c_scr[pl.ds(i * BR, BR), :] = c
        s_sum[...] += jnp.sum(c, axis=0, keepdims=True)
        s_sq[...] += jnp.sum(c * c, axis=0, keepdims=True)

    @pl.when(phase == 1)
    def _():
        c = c_scr[pl.ds(i * BR, BR), :]
        h = _bn_relu(c, s_sum, s_sq, g_ref, be_ref)
        gids = lax.broadcasted_iota(jnp.int32, (NG, BR), 0)
        pmat = (bat_ref[...] == gids).astype(jnp.float32)
        psum[...] += jnp.dot(pmat, h, preferred_element_type=jnp.float32)
        pcnt[...] += jnp.sum(pmat, axis=1, keepdims=True)

        @pl.when(i == NBLK - 1)
        def _():
            pooled = psum[...] / jnp.maximum(pcnt[...], 1.0)
            z = jnp.maximum(jnp.dot(pooled, wl1_ref[...],
                                    preferred_element_type=jnp.float32)
                            + bl1_ref[...], 0.0)
            out_ref[...] = jnp.dot(z, wl2_ref[...],
                                   preferred_element_type=jnp.float32) \
                + bl2_ref[...]


def _tfinal(p, u, degp, b, g, be, bat, wl1, bl1, wl2, bl2):
    return pl.pallas_call(
        _tfinal_body,
        grid=(2, NBLK),
        in_specs=[
            pl.BlockSpec((BR, H), lambda p_, i: (i, 0)),
            pl.BlockSpec((BR, H), lambda p_, i: (i, 0)),
            pl.BlockSpec((2, BR, DEGW), lambda p_, i: (0, i, 0)),
            pl.BlockSpec((H,), lambda p_, i: (0,)),
            pl.BlockSpec((H,), lambda p_, i: (0,)),
            pl.BlockSpec((H,), lambda p_, i: (0,)),
            pl.BlockSpec((1, BR), lambda p_, i: (0, i)),
            pl.BlockSpec((H, H // 2), lambda p_, i: (0, 0)),
            pl.BlockSpec((H // 2,), lambda p_, i: (0,)),
            pl.BlockSpec((H // 2, OUT), lambda p_, i: (0, 0)),
            pl.BlockSpec((OUT,), lambda p_, i: (0,)),
        ],
        out_specs=pl.BlockSpec((NG, OUT), lambda p_, i: (0, 0)),
        out_shape=jax.ShapeDtypeStruct((NG, OUT), jnp.float32),
        scratch_shapes=[
            pltpu.VMEM((N_PAD, H), jnp.float32),
            pltpu.VMEM((1, H), jnp.float32),
            pltpu.VMEM((1, H), jnp.float32),
            pltpu.VMEM((NG, H), jnp.float32),
            pltpu.VMEM((NG, 1), jnp.float32),
        ],
    )(p, u, degp, b, g, be, bat, wl1, bl1, wl2, bl2)


def kernel(x, edge_index, batch, W1, b1, W2, b2, W3, b3, W4, b4,
           g1, be1, g2, be2, g3, be3, g4, be4, Wl1, bl1, Wl2, bl2):
    src = edge_index[0]
    dst = edge_index[1]
    pad = jnp.full((E_PAD - E,), N, dtype=jnp.int32)
    src_flat = jnp.concatenate([src, pad])
    dst_flat = jnp.concatenate([dst, pad])
    src3 = src_flat.reshape(16, RNDS, CHUNK)
    dst3 = dst_flat.reshape(16, RNDS, CHUNK)

    x_pad = jnp.pad(x, ((0, N_PAD - N), (0, 0)))
    bat = jnp.pad(batch, (0, N_PAD - N), constant_values=NG + 1).reshape(1, N_PAD)

    zeros_d = jnp.zeros((RPT, DEGW), jnp.float32)
    ones_d = jnp.zeros((CHUNK, DEGW), jnp.float32).at[:, 0].set(1.0)

    degp = _sc_degree(dst3, ones_d, zeros_d)

    u = _t0(x_pad, W1, degp)
    p = _sc_scatter(u, src3, dst3)
    u = _tmid(p, u, degp, b1, g1, be1, W2)
    p = _sc_scatter(u, src3, dst3)
    u = _tmid(p, u, degp, b2, g2, be2, W3)
    p = _sc_scatter(u, src3, dst3)
    u = _tmid(p, u, degp, b3, g3, be3, W4)
    p = _sc_scatter(u, src3, dst3)
    return _tfinal(p, u, degp, b4, g4, be4, bat, Wl1, bl1, Wl2, bl2)


# speedup vs baseline: 21.1434x; 1.0301x over previous
import functools

import jax
import jax.numpy as jnp
from jax import lax
from jax.experimental import pallas as pl
from jax.experimental.pallas import tpu as pltpu
from jax.experimental.pallas import tpu_sc as plsc

N = 10000
E = 320000
H = 128
HH = 64
NG = 64
OUT = 64

NW = 32
CHUNK = 128
ROUNDS = 80
E_PAD = NW * ROUNDS * CHUNK
N_PAD = 10240
RPT = N_PAD // 16
DEGW = 8

_mesh = plsc.VectorSubcoreMesh(core_axis_name="c", subcore_axis_name="s",
                               num_cores=2, num_subcores=16)


RNDS = 2 * ROUNDS
GRPS = RNDS // 8


@functools.partial(
    pl.kernel,
    out_type=jax.ShapeDtypeStruct((N_PAD, H), jnp.float32),
    mesh=_mesh,
    scratch_types=[
        pltpu.VMEM((2, 8, CHUNK), jnp.int32),
        pltpu.VMEM((2, 8, CHUNK), jnp.int32),
        pltpu.VMEM((CHUNK, HH), jnp.float32),
        pltpu.VMEM((CHUNK, HH), jnp.float32),
        pltpu.VMEM_SHARED((N_PAD, HH), jnp.float32),
        pltpu.VMEM_SHARED((N_PAD, HH), jnp.float32),
        pltpu.SemaphoreType.DMA,
        pltpu.SemaphoreType.DMA,
        pltpu.SemaphoreType.DMA,
        pltpu.SemaphoreType.DMA,
    ],
    compiler_params=pltpu.CompilerParams(use_tc_tiling_on_sc=False),
)
def _sc_scatter(u_hbm, src_hbm, dst_hbm, out_hbm,
                src_blk, dst_blk, r0, r1, ustage, acc, s0, s1, si, di):
    cid = lax.axis_index("c")
    sid = lax.axis_index("s")

    bufs = (r0, r1)
    sems = (s0, s1)
    zvec = jnp.zeros((16,), jnp.float32)

    def full_pass():
        def zbody(i, carry):
            for j in range(HH // 16):
                r0[i, pl.ds(j * 16, 16)] = zvec
            return carry

        pltpu.async_copy(u_hbm.at[pl.ds(sid * RPT, RPT), pl.ds(cid * HH, HH)],
                         ustage.at[pl.ds(sid * RPT, RPT)], si)
        lax.fori_loop(0, CHUNK, zbody, 0)
        for q in range(RPT // CHUNK):
            pltpu.sync_copy(r0, acc.at[pl.ds(sid * RPT + q * CHUNK, CHUNK)])
        pltpu.make_async_copy(
            u_hbm.at[pl.ds(sid * RPT, RPT), pl.ds(cid * HH, HH)],
            ustage.at[pl.ds(sid * RPT, RPT)], si).wait()
        plsc.subcore_barrier()

        def idx_fire(g, q):
            pltpu.async_copy(src_hbm.at[sid, pl.ds(g * 8, 8)],
                             src_blk.at[q], si)
            pltpu.async_copy(dst_hbm.at[sid, pl.ds(g * 8, 8)],
                             dst_blk.at[q], di)

        def idx_wait(g, q):
            pltpu.make_async_copy(src_hbm.at[sid, pl.ds(g * 8, 8)],
                                  src_blk.at[q], si).wait()
            pltpu.make_async_copy(dst_hbm.at[sid, pl.ds(g * 8, 8)],
                                  dst_blk.at[q], di).wait()

        def gather(k, q, r, b):
            pltpu.async_copy(ustage.at[src_blk.at[q, r]], bufs[b], sems[b])

        def wait_gather(k, q, r, b):
            pltpu.make_async_copy(ustage.at[src_blk.at[q, r]], bufs[b],
                                  sems[b]).wait()

        def scatter(k, q, r, b):
            pltpu.async_copy(bufs[b], acc.at[dst_blk.at[q, r]], sems[b],
                             add=True)

        def wait_scatter(k, q, r, b):
            pltpu.make_async_copy(bufs[b], acc.at[dst_blk.at[q, r]],
                                  sems[b]).wait()

        pltpu.sync_copy(src_hbm.at[sid, pl.ds(0, 8)], src_blk.at[0])
        pltpu.sync_copy(dst_hbm.at[sid, pl.ds(0, 8)], dst_blk.at[0])
        idx_fire(1, 1)
        gather(0, 0, 0, 0)

        def body(i, carry):
            for kk in range(16):
                k = 16 * i + kk
                q = kk // 8
                r = kk % 8
                b = kk % 2
                wait_gather(k, q, r, b)
                scatter(k, q, r, b)

                if kk == 1:
                    @pl.when(i >= 1)
                    def _(i=i):
                        idx_fire(2 * i + 1, 1)
                if kk == 9:
                    @pl.when(i < (GRPS - 2) // 2)
                    def _(i=i):
                        idx_fire(2 * i + 2, 0)
                if kk == 7:
                    idx_wait(2 * i + 1, 1)

                nq = (kk + 1) // 8 % 2
                nr = (kk + 1) % 8
                nb = (kk + 1) % 2
                if kk == 15:
                    @pl.when(i < (RNDS // 16) - 1)
                    def _(i=i, k=k):
                        idx_wait(2 * i + 2, 0)
                        wait_scatter(k - 1, 1, 7, nb)
                        gather(k + 1, 0, 0, nb)
                else:
                    @pl.when(k >= 1)
                    def _(k=k, q=q, r=r, nb=nb):
                        wait_scatter(k - 1, q, r, nb)

                    gather(k + 1, nq, nr, nb)

            return carry

        lax.fori_loop(0, RNDS // 16, body, 0)
        wait_scatter(RNDS - 2, 1, 6, 0)
        wait_scatter(RNDS - 1, 1, 7, 1)
        plsc.subcore_barrier()
        pltpu.sync_copy(acc.at[pl.ds(sid * RPT, RPT)],
                        out_hbm.at[pl.ds(sid * RPT, RPT), pl.ds(cid * HH, HH)])

    full_pass()


@functools.partial(
    pl.kernel,
    out_type=jax.ShapeDtypeStruct((N_PAD, H), jnp.float32),
    mesh=_mesh,
    scratch_types=[
        pltpu.VMEM((ROUNDS, CHUNK), jnp.int32),
        pltpu.VMEM((CHUNK, DEGW), jnp.float32),
        pltpu.VMEM_SHARED((N_PAD, DEGW), jnp.float32),
        pltpu.SemaphoreType.DMA,
    ],
    compiler_params=pltpu.CompilerParams(use_tc_tiling_on_sc=False),
)
def _sc_degree(dst_hbm, ones_hbm, zeros_hbm, out_hbm, dst_v, ones_v, acc, sem):
    cid = lax.axis_index("c")
    sid = lax.axis_index("s")
    pltpu.sync_copy(dst_hbm.at[sid, pl.ds(cid * ROUNDS, ROUNDS)], dst_v)
    pltpu.sync_copy(ones_hbm, ones_v)
    pltpu.sync_copy(zeros_hbm, acc.at[pl.ds(sid * RPT, RPT)])
    plsc.subcore_barrier()

    def body(i, carry):
        for t in range(8):
            pltpu.async_copy(ones_v, acc.at[dst_v.at[8 * i + t]], sem,
                             add=True)
        for t in range(8):
            pltpu.make_async_copy(ones_v, acc.at[dst_v.at[8 * i + t]],
                                  sem).wait()
        return carry

    lax.fori_loop(0, ROUNDS // 8, body, 0)
    plsc.subcore_barrier()
    pltpu.sync_copy(acc.at[pl.ds(sid * RPT, RPT)],
                    out_hbm.at[pl.ds(sid * RPT, RPT),
                               pl.ds(cid * DEGW, DEGW)])


BR = 2560
NBLK = N_PAD // BR


def _dinv_of(degp_ref):
    deg = degp_ref[:, 0:1] + degp_ref[:, DEGW:DEGW + 1] + 1.0
    return lax.rsqrt(deg)


def _t0_body(x_ref, w_ref, degp_ref, u_ref):
    dinv = _dinv_of(degp_ref)
    u_ref[...] = jnp.dot(x_ref[...], w_ref[...],
                         preferred_element_type=jnp.float32) * dinv


def _t0(x_pad, w, degp):
    return pl.pallas_call(
        _t0_body,
        grid=(NBLK,),
        in_specs=[
            pl.BlockSpec((BR, H), lambda i: (i, 0)),
            pl.BlockSpec((H, H), lambda i: (0, 0)),
            pl.BlockSpec((BR, H), lambda i: (i, 0)),
        ],
        out_specs=pl.BlockSpec((BR, H), lambda i: (i, 0)),
        out_shape=jax.ShapeDtypeStruct((N_PAD, H), jnp.float32),
    )(x_pad, w, degp)


def _combine_c(p_ref, u_ref, degp_ref, b_ref, i):
    dinv = _dinv_of(degp_ref)
    c = (p_ref[...] + u_ref[...]) * dinv + b_ref[...]
    rows = lax.broadcasted_iota(jnp.int32, (BR, 1), 0) + i * BR
    return jnp.where(rows < N, c, 0.0)


def _bn_relu(c, s_sum, s_sq, g_ref, be_ref):
    mu = s_sum[...] / N
    var = s_sq[...] / N - mu * mu
    return jnp.maximum(g_ref[...] * (c - mu) * lax.rsqrt(var + 1e-5)
                       + be_ref[...], 0.0)


def _tmid_body(p_ref, u_ref, degp_ref, b_ref, g_ref, be_ref,
               wn_ref, o_ref, c_scr, s_sum, s_sq):
    phase = pl.program_id(0)
    i = pl.program_id(1)

    @pl.when(phase == 0)
    def _():
        @pl.when(i == 0)
        def _():
            s_sum[...] = jnp.zeros_like(s_sum)
            s_sq[...] = jnp.zeros_like(s_sq)

        c = _combine_c(p_ref, u_ref, degp_ref, b_ref, i)
        c_scr[pl.ds(i * BR, BR), :] = c
        s_sum[...] += jnp.sum(c, axis=0, keepdims=True)
        s_sq[...] += jnp.sum(c * c, axis=0, keepdims=True)

    @pl.when(phase == 1)
    def _():
        c = c_scr[pl.ds(i * BR, BR), :]
        h = _bn_relu(c, s_sum, s_sq, g_ref, be_ref)
        rows = lax.broadcasted_iota(jnp.int32, (BR, 1), 0) + i * BR
        h = jnp.where(rows < N, h, 0.0)
        dinv = _dinv_of(degp_ref)
        o_ref[...] = jnp.dot(h, wn_ref[...],
                             preferred_element_type=jnp.float32) * dinv


def _tmid(p, u, degp, b, g, be, wn):
    return pl.pallas_call(
        _tmid_body,
        grid=(2, NBLK),
        in_specs=[
            pl.BlockSpec((BR, H), lambda p_, i: (i, 0)),
            pl.BlockSpec((BR, H), lambda p_, i: (i, 0)),
            pl.BlockSpec((BR, H), lambda p_, i: (i, 0)),
            pl.BlockSpec((H,), lambda p_, i: (0,)),
            pl.BlockSpec((H,), lambda p_, i: (0,)),
            pl.BlockSpec((H,), lambda p_, i: (0,)),
            pl.BlockSpec((H, H), lambda p_, i: (0, 0)),
        ],
        out_specs=pl.BlockSpec((BR, H), lambda p_, i: (i, 0)),
        out_shape=jax.ShapeDtypeStruct((N_PAD, H), jnp.float32),
        scratch_shapes=[
            pltpu.VMEM((N_PAD, H), jnp.float32),
            pltpu.VMEM((1, H), jnp.float32),
            pltpu.VMEM((1, H), jnp.float32),
        ],
    )(p, u, degp, b, g, be, wn)


def _tfinal_body(p_ref, u_ref, degp_ref, b_ref, g_ref,
                 be_ref, bat_ref, wl1_ref, bl1_ref, wl2_ref, bl2_ref, out_ref,
                 c_scr, s_sum, s_sq, psum, pcnt):
    phase = pl.program_id(0)
    i = pl.program_id(1)

    @pl.when(phase == 0)
    def _():
        @pl.when(i == 0)
        def _():
            s_sum[...] = jnp.zeros_like(s_sum)
            s_sq[...] = jnp.zeros_like(s_sq)
            psum[...] = jnp.zeros_like(psum)
            pcnt[...] = jnp.zeros_like(pcnt)

        c = _combine_c(p_ref, u_ref, degp_ref, b_ref, i)
        c_scr[pl.ds(i * BR, BR), :] = c
        s_sum[...] += jnp.sum(c, axis=0, keepdims=True)
        s_sq[...] += jnp.sum(c * c, axis=0, keepdims=True)

    @pl.when(phase == 1)
    def _():
        c = c_scr[pl.ds(i * BR, BR), :]
        h = _bn_relu(c, s_sum, s_sq, g_ref, be_ref)
        gids = lax.broadcasted_iota(jnp.int32, (NG, BR), 0)
        pmat = (bat_ref[...] == gids).astype(jnp.float32)
        psum[...] += jnp.dot(pmat, h, preferred_element_type=jnp.float32)
        pcnt[...] += jnp.sum(pmat, axis=1, keepdims=True)

        @pl.when(i == NBLK - 1)
        def _():
            pooled = psum[...] / jnp.maximum(pcnt[...], 1.0)
            z = jnp.maximum(jnp.dot(pooled, wl1_ref[...],
                                    preferred_element_type=jnp.float32)
                            + bl1_ref[...], 0.0)
            out_ref[...] = jnp.dot(z, wl2_ref[...],
                                   preferred_element_type=jnp.float32) \
                + bl2_ref[...]


def _tfinal(p, u, degp, b, g, be, bat, wl1, bl1, wl2, bl2):
    return pl.pallas_call(
        _tfinal_body,
        grid=(2, NBLK),
        in_specs=[
            pl.BlockSpec((BR, H), lambda p_, i: (i, 0)),
            pl.BlockSpec((BR, H), lambda p_, i: (i, 0)),
            pl.BlockSpec((BR, H), lambda p_, i: (i, 0)),
            pl.BlockSpec((H,), lambda p_, i: (0,)),
            pl.BlockSpec((H,), lambda p_, i: (0,)),
            pl.BlockSpec((H,), lambda p_, i: (0,)),
            pl.BlockSpec((1, BR), lambda p_, i: (0, i)),
            pl.BlockSpec((H, H // 2), lambda p_, i: (0, 0)),
            pl.BlockSpec((H // 2,), lambda p_, i: (0,)),
            pl.BlockSpec((H // 2, OUT), lambda p_, i: (0, 0)),
            pl.BlockSpec((OUT,), lambda p_, i: (0,)),
        ],
        out_specs=pl.BlockSpec((NG, OUT), lambda p_, i: (0, 0)),
        out_shape=jax.ShapeDtypeStruct((NG, OUT), jnp.float32),
        scratch_shapes=[
            pltpu.VMEM((N_PAD, H), jnp.float32),
            pltpu.VMEM((1, H), jnp.float32),
            pltpu.VMEM((1, H), jnp.float32),
            pltpu.VMEM((NG, H), jnp.float32),
            pltpu.VMEM((NG, 1), jnp.float32),
        ],
    )(p, u, degp, b, g, be, bat, wl1, bl1, wl2, bl2)


def kernel(x, edge_index, batch, W1, b1, W2, b2, W3, b3, W4, b4,
           g1, be1, g2, be2, g3, be3, g4, be4, Wl1, bl1, Wl2, bl2):
    src = edge_index[0]
    dst = edge_index[1]
    pad = jnp.full((E_PAD - E,), N, dtype=jnp.int32)
    src_flat = jnp.concatenate([src, pad])
    dst_flat = jnp.concatenate([dst, pad])
    src3 = src_flat.reshape(16, RNDS, CHUNK)
    dst3 = dst_flat.reshape(16, RNDS, CHUNK)

    x_pad = jnp.pad(x, ((0, N_PAD - N), (0, 0)))
    bat = jnp.pad(batch, (0, N_PAD - N), constant_values=NG + 1).reshape(1, N_PAD)

    zeros_d = jnp.zeros((RPT, DEGW), jnp.float32)
    ones_d = jnp.zeros((CHUNK, DEGW), jnp.float32).at[:, 0].set(1.0)

    degp = _sc_degree(dst3, ones_d, zeros_d)

    u = _t0(x_pad, W1, degp)
    p = _sc_scatter(u, src3, dst3)
    u = _tmid(p, u, degp, b1, g1, be1, W2)
    p = _sc_scatter(u, src3, dst3)
    u = _tmid(p, u, degp, b2, g2, be2, W3)
    p = _sc_scatter(u, src3, dst3)
    u = _tmid(p, u, degp, b3, g3, be3, W4)
    p = _sc_scatter(u, src3, dst3)
    return _tfinal(p, u, degp, b4, g4, be4, bat, Wl1, bl1, Wl2, bl2)


# speedup vs baseline: 21.2140x; 1.0033x over previous
import functools

import jax
import jax.numpy as jnp
from jax import lax
from jax.experimental import pallas as pl
from jax.experimental.pallas import tpu as pltpu
from jax.experimental.pallas import tpu_sc as plsc

N = 10000
E = 320000
H = 128
HH = 64
NG = 64
OUT = 64

NW = 32
CHUNK = 128
ROUNDS = 80
E_PAD = NW * ROUNDS * CHUNK
N_PAD = 10240
RPT = N_PAD // 16
DEGW = 8

_mesh = plsc.VectorSubcoreMesh(core_axis_name="c", subcore_axis_name="s",
                               num_cores=2, num_subcores=16)


RNDS = 2 * ROUNDS
GRPS = RNDS // 8


@functools.partial(
    pl.kernel,
    out_type=jax.ShapeDtypeStruct((N_PAD, H), jnp.float32),
    mesh=_mesh,
    scratch_types=[
        pltpu.VMEM((2, 8, CHUNK), jnp.int32),
        pltpu.VMEM((2, 8, CHUNK), jnp.int32),
        pltpu.VMEM((CHUNK, HH), jnp.float32),
        pltpu.VMEM((CHUNK, HH), jnp.float32),
        pltpu.VMEM_SHARED((N_PAD, HH), jnp.float32),
        pltpu.VMEM_SHARED((N_PAD, HH), jnp.float32),
        pltpu.SemaphoreType.DMA,
        pltpu.SemaphoreType.DMA,
        pltpu.SemaphoreType.DMA,
        pltpu.SemaphoreType.DMA,
    ],
    compiler_params=pltpu.CompilerParams(use_tc_tiling_on_sc=False),
)
def _sc_scatter(u_hbm, src_hbm, dst_hbm, out_hbm,
                src_blk, dst_blk, r0, r1, ustage, acc, s0, s1, si, di):
    cid = lax.axis_index("c")
    sid = lax.axis_index("s")

    bufs = (r0, r1)
    sems = (s0, s1)
    zvec = jnp.zeros((16,), jnp.float32)

    def full_pass():
        def zbody(i, carry):
            for j in range(HH // 16):
                r0[i, pl.ds(j * 16, 16)] = zvec
            return carry

        pltpu.async_copy(u_hbm.at[pl.ds(sid * RPT, RPT), pl.ds(cid * HH, HH)],
                         ustage.at[pl.ds(sid * RPT, RPT)], si)
        lax.fori_loop(0, CHUNK, zbody, 0)
        for q in range(RPT // CHUNK):
            pltpu.sync_copy(r0, acc.at[pl.ds(sid * RPT + q * CHUNK, CHUNK)])
        pltpu.make_async_copy(
            u_hbm.at[pl.ds(sid * RPT, RPT), pl.ds(cid * HH, HH)],
            ustage.at[pl.ds(sid * RPT, RPT)], si).wait()
        plsc.subcore_barrier()

        def idx_fire(g, q):
            pltpu.async_copy(src_hbm.at[sid, pl.ds(g * 8, 8)],
                             src_blk.at[q], si)
            pltpu.async_copy(dst_hbm.at[sid, pl.ds(g * 8, 8)],
                             dst_blk.at[q], di)

        def idx_wait(g, q):
            pltpu.make_async_copy(src_hbm.at[sid, pl.ds(g * 8, 8)],
                                  src_blk.at[q], si).wait()
            pltpu.make_async_copy(dst_hbm.at[sid, pl.ds(g * 8, 8)],
                                  dst_blk.at[q], di).wait()

        def gather(k, q, r, b):
            pltpu.async_copy(ustage.at[src_blk.at[q, r]], bufs[b], sems[b])

        def wait_gather(k, q, r, b):
            pltpu.make_async_copy(ustage.at[src_blk.at[q, r]], bufs[b],
                                  sems[b]).wait()

        def scatter(k, q, r, b):
            pltpu.async_copy(bufs[b], acc.at[dst_blk.at[q, r]], sems[b],
                             add=True)

        def wait_scatter(k, q, r, b):
            pltpu.make_async_copy(bufs[b], acc.at[dst_blk.at[q, r]],
                                  sems[b]).wait()

        pltpu.sync_copy(src_hbm.at[sid, pl.ds(0, 8)], src_blk.at[0])
        pltpu.sync_copy(dst_hbm.at[sid, pl.ds(0, 8)], dst_blk.at[0])
        idx_fire(1, 1)
        gather(0, 0, 0, 0)

        def body(i, carry):
            for kk in range(16):
                k = 16 * i + kk
                q = kk // 8
                r = kk % 8
                b = kk % 2
                wait_gather(k, q, r, b)
                scatter(k, q, r, b)

                if kk == 1:
                    @pl.when(i >= 1)
                    def _(i=i):
                        idx_fire(2 * i + 1, 1)
                if kk == 9:
                    @pl.when(i < (GRPS - 2) // 2)
                    def _(i=i):
                        idx_fire(2 * i + 2, 0)
                if kk == 7:
                    idx_wait(2 * i + 1, 1)

                nq = (kk + 1) // 8 % 2
                nr = (kk + 1) % 8
                nb = (kk + 1) % 2
                if kk == 15:
                    @pl.when(i < (RNDS // 16) - 1)
                    def _(i=i, k=k):
                        idx_wait(2 * i + 2, 0)
                        wait_scatter(k - 1, 1, 7, nb)
                        gather(k + 1, 0, 0, nb)
                else:
                    @pl.when(k >= 1)
                    def _(k=k, q=q, r=r, nb=nb):
                        wait_scatter(k - 1, q, r, nb)

                    gather(k + 1, nq, nr, nb)

            return carry

        lax.fori_loop(0, RNDS // 16, body, 0)
        wait_scatter(RNDS - 2, 1, 6, 0)
        wait_scatter(RNDS - 1, 1, 7, 1)
        plsc.subcore_barrier()
        pltpu.sync_copy(acc.at[pl.ds(sid * RPT, RPT)],
                        out_hbm.at[pl.ds(sid * RPT, RPT), pl.ds(cid * HH, HH)])

    full_pass()


@functools.partial(
    pl.kernel,
    out_type=jax.ShapeDtypeStruct((N_PAD, H), jnp.float32),
    mesh=_mesh,
    scratch_types=[
        pltpu.VMEM((ROUNDS, CHUNK), jnp.int32),
        pltpu.VMEM((CHUNK, DEGW), jnp.float32),
        pltpu.VMEM_SHARED((N_PAD, DEGW), jnp.float32),
        pltpu.SemaphoreType.DMA,
    ],
    compiler_params=pltpu.CompilerParams(use_tc_tiling_on_sc=False),
)
def _sc_degree(dst_hbm, ones_hbm, zeros_hbm, out_hbm, dst_v, ones_v, acc, sem):
    cid = lax.axis_index("c")
    sid = lax.axis_index("s")
    pltpu.sync_copy(dst_hbm.at[sid, pl.ds(cid * ROUNDS, ROUNDS)], dst_v)
    pltpu.sync_copy(ones_hbm, ones_v)
    pltpu.sync_copy(zeros_hbm, acc.at[pl.ds(sid * RPT, RPT)])
    plsc.subcore_barrier()

    def body(i, carry):
        for t in range(8):
            pltpu.async_copy(ones_v, acc.at[dst_v.at[8 * i + t]], sem,
                             add=True)
        for t in range(8):
            pltpu.make_async_copy(ones_v, acc.at[dst_v.at[8 * i + t]],
                                  sem).wait()
        return carry

    lax.fori_loop(0, ROUNDS // 8, body, 0)
    plsc.subcore_barrier()
    pltpu.sync_copy(acc.at[pl.ds(sid * RPT, RPT)],
                    out_hbm.at[pl.ds(sid * RPT, RPT),
                               pl.ds(cid * DEGW, DEGW)])


BR = 5120
NBLK = N_PAD // BR


def _dinv_of(degp_ref):
    deg = degp_ref[:, 0:1] + degp_ref[:, DEGW:DEGW + 1] + 1.0
    return lax.rsqrt(deg)


def _t0_body(x_ref, w_ref, degp_ref, u_ref):
    dinv = _dinv_of(degp_ref)
    u_ref[...] = jnp.dot(x_ref[...], w_ref[...],
                         preferred_element_type=jnp.float32) * dinv


def _t0(x_pad, w, degp):
    return pl.pallas_call(
        _t0_body,
        grid=(NBLK,),
        in_specs=[
            pl.BlockSpec((BR, H), lambda i: (i, 0)),
            pl.BlockSpec((H, H), lambda i: (0, 0)),
            pl.BlockSpec((BR, H), lambda i: (i, 0)),
        ],
        out_specs=pl.BlockSpec((BR, H), lambda i: (i, 0)),
        out_shape=jax.ShapeDtypeStruct((N_PAD, H), jnp.float32),
    )(x_pad, w, degp)


def _combine_c(p_ref, u_ref, degp_ref, b_ref, i):
    dinv = _dinv_of(degp_ref)
    c = (p_ref[...] + u_ref[...]) * dinv + b_ref[...]
    rows = lax.broadcasted_iota(jnp.int32, (BR, 1), 0) + i * BR
    return jnp.where(rows < N, c, 0.0)


def _bn_relu(c, s_sum, s_sq, g_ref, be_ref):
    mu = s_sum[...] / N
    var = s_sq[...] / N - mu * mu
    return jnp.maximum(g_ref[...] * (c - mu) * lax.rsqrt(var + 1e-5)
                       + be_ref[...], 0.0)


def _tmid_body(p_ref, u_ref, degp_ref, b_ref, g_ref, be_ref,
               wn_ref, o_ref, c_scr, s_sum, s_sq):
    phase = pl.program_id(0)
    i = pl.program_id(1)

    @pl.when(phase == 0)
    def _():
        @pl.when(i == 0)
        def _():
            s_sum[...] = jnp.zeros_like(s_sum)
            s_sq[...] = jnp.zeros_like(s_sq)

        c = _combine_c(p_ref, u_ref, degp_ref, b_ref, i)
        c_scr[pl.ds(i * BR, BR), :] = c
        s_sum[...] += jnp.sum(c, axis=0, keepdims=True)
        s_sq[...] += jnp.sum(c * c, axis=0, keepdims=True)

    @pl.when(phase == 1)
    def _():
        c = c_scr[pl.ds(i * BR, BR), :]
        h = _bn_relu(c, s_sum, s_sq, g_ref, be_ref)
        rows = lax.broadcasted_iota(jnp.int32, (BR, 1), 0) + i * BR
        h = jnp.where(rows < N, h, 0.0)
        dinv = _dinv_of(degp_ref)
        o_ref[...] = jnp.dot(h, wn_ref[...],
                             preferred_element_type=jnp.float32) * dinv


def _tmid(p, u, degp, b, g, be, wn):
    return pl.pallas_call(
        _tmid_body,
        grid=(2, NBLK),
        in_specs=[
            pl.BlockSpec((BR, H), lambda p_, i: (i, 0)),
            pl.BlockSpec((BR, H), lambda p_, i: (i, 0)),
            pl.BlockSpec((BR, H), lambda p_, i: (i, 0)),
            pl.BlockSpec((H,), lambda p_, i: (0,)),
            pl.BlockSpec((H,), lambda p_, i: (0,)),
            pl.BlockSpec((H,), lambda p_, i: (0,)),
            pl.BlockSpec((H, H), lambda p_, i: (0, 0)),
        ],
        out_specs=pl.BlockSpec((BR, H), lambda p_, i: (i, 0)),
        out_shape=jax.ShapeDtypeStruct((N_PAD, H), jnp.float32),
        scratch_shapes=[
            pltpu.VMEM((N_PAD, H), jnp.float32),
            pltpu.VMEM((1, H), jnp.float32),
            pltpu.VMEM((1, H), jnp.float32),
        ],
    )(p, u, degp, b, g, be, wn)


def _tfinal_body(p_ref, u_ref, degp_ref, b_ref, g_ref,
                 be_ref, bat_ref, wl1_ref, bl1_ref, wl2_ref, bl2_ref, out_ref,
                 c_scr, s_sum, s_sq, psum, pcnt):
    phase = pl.program_id(0)
    i = pl.program_id(1)

    @pl.when(phase == 0)
    def _():
        @pl.when(i == 0)
        def _():
            s_sum[...] = jnp.zeros_like(s_sum)
            s_sq[...] = jnp.zeros_like(s_sq)
            psum[...] = jnp.zeros_like(psum)
            pcnt[...] = jnp.zeros_like(pcnt)

        c = _combine_c(p_ref, u_ref, degp_ref, b_ref, i)
        c_scr[pl.ds(i * BR, BR), :] = c
        s_sum[...] += jnp.sum(c, axis=0, keepdims=True)
        s_sq[...] += jnp.sum(c * c, axis=0, keepdims=True)

    @pl.when(phase == 1)
    def _():
        c = c_scr[pl.ds(i * BR, BR), :]
        h = _bn_relu(c, s_sum, s_sq, g_ref, be_ref)
        gids = lax.broadcasted_iota(jnp.int32, (NG, BR), 0)
        pmat = (bat_ref[...] == gids).astype(jnp.float32)
        psum[...] += jnp.dot(pmat, h, preferred_element_type=jnp.float32)
        pcnt[...] += jnp.sum(pmat, axis=1, keepdims=True)

        @pl.when(i == NBLK - 1)
        def _():
            pooled = psum[...] / jnp.maximum(pcnt[...], 1.0)
            z = jnp.maximum(jnp.dot(pooled, wl1_ref[...],
                                    preferred_element_type=jnp.float32)
                            + bl1_ref[...], 0.0)
            out_ref[...] = jnp.dot(z, wl2_ref[...],
                                   preferred_element_type=jnp.float32) \
                + bl2_ref[...]


def _tfinal(p, u, degp, b, g, be, bat, wl1, bl1, wl2, bl2):
    return pl.pallas_call(
        _tfinal_body,
        grid=(2, NBLK),
        in_specs=[
            pl.BlockSpec((BR, H), lambda p_, i: (i, 0)),
            pl.BlockSpec((BR, H), lambda p_, i: (i, 0)),
            pl.BlockSpec((BR, H), lambda p_, i: (i, 0)),
            pl.BlockSpec((H,), lambda p_, i: (0,)),
            pl.BlockSpec((H,), lambda p_, i: (0,)),
            pl.BlockSpec((H,), lambda p_, i: (0,)),
            pl.BlockSpec((1, BR), lambda p_, i: (0, i)),
            pl.BlockSpec((H, H // 2), lambda p_, i: (0, 0)),
            pl.BlockSpec((H // 2,), lambda p_, i: (0,)),
            pl.BlockSpec((H // 2, OUT), lambda p_, i: (0, 0)),
            pl.BlockSpec((OUT,), lambda p_, i: (0,)),
        ],
        out_specs=pl.BlockSpec((NG, OUT), lambda p_, i: (0, 0)),
        out_shape=jax.ShapeDtypeStruct((NG, OUT), jnp.float32),
        scratch_shapes=[
            pltpu.VMEM((N_PAD, H), jnp.float32),
            pltpu.VMEM((1, H), jnp.float32),
            pltpu.VMEM((1, H), jnp.float32),
            pltpu.VMEM((NG, H), jnp.float32),
            pltpu.VMEM((NG, 1), jnp.float32),
        ],
    )(p, u, degp, b, g, be, bat, wl1, bl1, wl2, bl2)


def kernel(x, edge_index, batch, W1, b1, W2, b2, W3, b3, W4, b4,
           g1, be1, g2, be2, g3, be3, g4, be4, Wl1, bl1, Wl2, bl2):
    src = edge_index[0]
    dst = edge_index[1]
    pad = jnp.full((E_PAD - E,), N, dtype=jnp.int32)
    src_flat = jnp.concatenate([src, pad])
    dst_flat = jnp.concatenate([dst, pad])
    src3 = src_flat.reshape(16, RNDS, CHUNK)
    dst3 = dst_flat.reshape(16, RNDS, CHUNK)

    x_pad = jnp.pad(x, ((0, N_PAD - N), (0, 0)))
    bat = jnp.pad(batch, (0, N_PAD - N), constant_values=NG + 1).reshape(1, N_PAD)

    zeros_d = jnp.zeros((RPT, DEGW), jnp.float32)
    ones_d = jnp.zeros((CHUNK, DEGW), jnp.float32).at[:, 0].set(1.0)

    degp = _sc_degree(dst3, ones_d, zeros_d)

    u = _t0(x_pad, W1, degp)
    p = _sc_scatter(u, src3, dst3)
    u = _tmid(p, u, degp, b1, g1, be1, W2)
    p = _sc_scatter(u, src3, dst3)
    u = _tmid(p, u, degp, b2, g2, be2, W3)
    p = _sc_scatter(u, src3, dst3)
    u = _tmid(p, u, degp, b3, g3, be3, W4)
    p = _sc_scatter(u, src3, dst3)
    return _tfinal(p, u, degp, b4, g4, be4, bat, Wl1, bl1, Wl2, bl2)
